# baseline (speedup 1.0000x reference)
import functools

import jax
import jax.numpy as jnp
from jax import lax
from jax.experimental import pallas as pl
from jax.experimental.pallas import tpu as pltpu
from jax.experimental.pallas import tpu_sc as plsc

E = 16
TOP_K = 2
HIDDEN = 1024
INTER = 512
T = 2048
NEG_INF = float("-inf")
HH = HIDDEN // 2
MASK_HI = -65536

BLK = 512
NB = (T * TOP_K) // BLK + (E - 1)
GROUP_ROWS = NB * BLK
CHUNK = 256
NW = 32
TPW = T // NW
CC = 32
NCH = TPW // CC


def _router_a_body(x_ref, gate_ref, oh_ref, wts_ref, xp_ref):
    x = x_ref[...]
    logits = lax.dot_general(x, gate_ref[...], (((1,), (1,)), ((), ())),
                             preferred_element_type=jnp.float32)
    ii = lax.broadcasted_iota(jnp.int32, (CHUNK, E), 1)
    m1 = jnp.max(logits, axis=-1, keepdims=True)
    i1 = jnp.min(jnp.where(logits == m1, ii, E), axis=-1, keepdims=True)
    l2 = jnp.where(ii == i1, NEG_INF, logits)
    m2 = jnp.max(l2, axis=-1, keepdims=True)
    i2 = jnp.min(jnp.where(l2 == m2, ii, E), axis=-1, keepdims=True)
    r = jnp.exp(m2 - m1)
    w1 = 1.0 / (1.0 + r)
    w2 = 1.0 - w1
    wts_ref[0] = jnp.broadcast_to(w1, (CHUNK, 16))
    wts_ref[1] = jnp.broadcast_to(w2, (CHUNK, 16))
    oh_ref[0] = (ii == i1).astype(jnp.float32)
    oh_ref[1] = (ii == i2).astype(jnp.float32)

    lo = lax.bitcast_convert_type(x[:, :HH], jnp.int32) + 32768
    hi = lax.bitcast_convert_type(x[:, HH:], jnp.int32) + 32768
    xp_ref[...] = lax.shift_right_logical(lo, 16) | (hi & MASK_HI)


def _router_b_body(oh_ref, dest_ref, meta_ref):
    oh0 = oh_ref[0]
    oh1 = oh_ref[1]

    rr = lax.broadcasted_iota(jnp.int32, (CHUNK, CHUNK), 0)
    cc = lax.broadcasted_iota(jnp.int32, (CHUNK, CHUNK), 1)
    ltri = (rr > cc).astype(jnp.float32)

    def _excl_cumsum(oh):
        parts = []
        off = jnp.zeros((1, E), jnp.float32)
        for c in range(T // CHUNK):
            blk = oh[c * CHUNK:(c + 1) * CHUNK, :]
            exc = lax.dot_general(ltri, blk, (((1,), (0,)), ((), ())),
                                  preferred_element_type=jnp.float32)
            parts.append(exc + off)
            off = off + jnp.sum(blk, axis=0, keepdims=True)
        return jnp.concatenate(parts, axis=0), off

    exc0, cnt0 = _excl_cumsum(oh0)
    exc1, cnt1 = _excl_cumsum(oh1)
    counts = cnt0 + cnt1
    nb = jnp.floor((counts + (BLK - 1)) / BLK)
    er = lax.broadcasted_iota(jnp.int32, (E, E), 0)
    ec = lax.broadcasted_iota(jnp.int32, (E, E), 1)
    eutri = (er < ec).astype(jnp.float32)
    bs = lax.dot_general(nb, eutri, (((1,), (0,)), ((), ())),
                         preferred_element_type=jnp.float32)
    total = jnp.sum(nb)

    rank0 = jnp.sum(oh0 * exc0, axis=1)
    rank1 = jnp.sum(oh1 * (cnt0 + exc1), axis=1)
    base0 = jnp.sum(oh0 * bs, axis=1) * BLK
    base1 = jnp.sum(oh1 * bs, axis=1) * BLK
    dest_ref[0, :] = (base0 + rank0).astype(jnp.int32)
    dest_ref[1, :] = (base1 + rank1).astype(jnp.int32)

    bb = lax.broadcasted_iota(jnp.int32, (64, E), 0).astype(jnp.float32)
    emap = jnp.sum((bs <= bb).astype(jnp.int32), axis=1) - 1
    bidx = lax.broadcasted_iota(jnp.int32, (64, 1), 0).astype(jnp.float32)[:, 0]
    active = (bidx < total).astype(jnp.int32)
    meta_ref[0, :] = jnp.clip(emap, 0, E - 1)
    meta_ref[1, :] = active
    meta_ref[2, :] = jnp.minimum(bidx, total - 1.0).astype(jnp.int32)


def _router_call(x, gate_w):
    oh, wts, xp = pl.pallas_call(
        _router_a_body,
        grid=(T // CHUNK,),
        in_specs=[
            pl.BlockSpec((CHUNK, HIDDEN), lambda c: (c, 0)),
            pl.BlockSpec((E, HIDDEN), lambda c: (0, 0)),
        ],
        out_specs=(
            pl.BlockSpec((2, CHUNK, E), lambda c: (0, c, 0)),
            pl.BlockSpec((2, CHUNK, 16), lambda c: (0, c, 0)),
            pl.BlockSpec((CHUNK, HH), lambda c: (c, 0)),
        ),
        out_shape=(
            jax.ShapeDtypeStruct((2, T, E), jnp.float32),
            jax.ShapeDtypeStruct((2, T, 16), jnp.float32),
            jax.ShapeDtypeStruct((T, HH), jnp.int32),
        ),
    )(x, gate_w)
    dest, meta = pl.pallas_call(
        _router_b_body,
        out_shape=(
            jax.ShapeDtypeStruct((2, T), jnp.int32),
            jax.ShapeDtypeStruct((3, 64), jnp.int32),
        ),
    )(oh)
    return dest, meta, wts, xp


def _make_sc_scatter():
    mesh = plsc.VectorSubcoreMesh(core_axis_name="c", subcore_axis_name="s")

    @functools.partial(
        pl.kernel, mesh=mesh,
        out_type=jax.ShapeDtypeStruct((GROUP_ROWS, HH), jnp.int32),
        scratch_types=[
            pltpu.VMEM((TPW, HH), jnp.int32),
            pltpu.VMEM((TPW,), jnp.int32),
            pltpu.VMEM((TPW,), jnp.int32),
            pltpu.SemaphoreType.DMA,
            pltpu.SemaphoreType.DMA,
        ],
    )
    def k(xp_hbm, dest_hbm, xg_hbm, rows_v, idx0_v, idx1_v, sem_a, sem_b):
        wid = lax.axis_index("s") * 2 + lax.axis_index("c")
        base = wid * TPW
        pltpu.sync_copy(dest_hbm.at[0, pl.ds(base, TPW)], idx0_v)
        pltpu.sync_copy(dest_hbm.at[1, pl.ds(base, TPW)], idx1_v)
        pltpu.sync_copy(xp_hbm.at[pl.ds(base, TPW)], rows_v)
        c0 = pltpu.async_copy(rows_v, xg_hbm.at[idx0_v], sem_a)
        c1 = pltpu.async_copy(rows_v, xg_hbm.at[idx1_v], sem_b)
        c0.wait()
        c1.wait()

    return k


def _ffn_body(s_ref, xg_ref, wg_ref, wu_ref, wd_ref, yg_ref):
    b = pl.program_id(0)

    @pl.when(s_ref[1, b] == 1)
    def _compute():
        v = xg_ref[...]
        xb = jnp.concatenate(
            [lax.bitcast_convert_type(lax.shift_left(v, 16), jnp.float32),
             lax.bitcast_convert_type(v & MASK_HI, jnp.float32)], axis=1)
        g = lax.dot_general(xb, wg_ref[0], (((1,), (1,)), ((), ())),
                            preferred_element_type=jnp.float32)
        u = lax.dot_general(xb, wu_ref[0], (((1,), (1,)), ((), ())),
                            preferred_element_type=jnp.float32)
        h = (g * jax.nn.sigmoid(g)) * u
        y = lax.dot_general(h, wd_ref[0], (((1,), (1,)), ((), ())),
                            preferred_element_type=jnp.float32)
        ylo = lax.bitcast_convert_type(y[:, :HH], jnp.int32) + 32768
        yhi = lax.bitcast_convert_type(y[:, HH:], jnp.int32) + 32768
        yg_ref[...] = lax.shift_right_logical(ylo, 16) | (yhi & MASK_HI)


def _ffn_call(meta, xg, w_gate, w_up, w_down):
    grid_spec = pltpu.PrefetchScalarGridSpec(
        num_scalar_prefetch=1,
        grid=(NB,),
        in_specs=[
            pl.BlockSpec((BLK, HH), lambda b, s: (s[2, b], 0)),
            pl.BlockSpec((1, INTER, HIDDEN), lambda b, s: (s[0, b], 0, 0)),
            pl.BlockSpec((1, INTER, HIDDEN), lambda b, s: (s[0, b], 0, 0)),
            pl.BlockSpec((1, HIDDEN, INTER), lambda b, s: (s[0, b], 0, 0)),
        ],
        out_specs=pl.BlockSpec((BLK, HH), lambda b, s: (s[2, b], 0)),
    )
    return pl.pallas_call(
        _ffn_body,
        grid_spec=grid_spec,
        out_shape=jax.ShapeDtypeStruct((GROUP_ROWS, HH), jnp.int32),
    )(meta, xg, w_gate, w_up, w_down)


def _make_sc_gather():
    mesh = plsc.VectorSubcoreMesh(core_axis_name="c", subcore_axis_name="s")

    @functools.partial(
        pl.kernel, mesh=mesh,
        out_type=(
            jax.ShapeDtypeStruct((T, HH), jnp.int32),
            jax.ShapeDtypeStruct((T, HH), jnp.int32),
        ),
        scratch_types=[
            pltpu.VMEM((CC, HH), jnp.int32),
            pltpu.VMEM((CC, HH), jnp.int32),
            pltpu.VMEM((CC, HH), jnp.int32),
            pltpu.VMEM((CC, HH), jnp.int32),
            pltpu.VMEM((TPW,), jnp.int32),
            pltpu.VMEM((TPW,), jnp.int32),
            pltpu.VMEM((CC,), jnp.int32),
            pltpu.VMEM((CC,), jnp.int32),
            pltpu.VMEM((CC,), jnp.int32),
            pltpu.VMEM((CC,), jnp.int32),
            pltpu.SemaphoreType.DMA,
            pltpu.SemaphoreType.DMA,
            pltpu.SemaphoreType.DMA,
            pltpu.SemaphoreType.DMA,
        ],
    )
    def k(yg_hbm, dest_hbm, y0_hbm, y1_hbm,
          a0_v, a1_v, b0_v, b1_v, idx0_v, idx1_v,
          ia0_v, ia1_v, ib0_v, ib1_v,
          sem_a, sem_b, sem_wa, sem_wb):
        wid = lax.axis_index("s") * 2 + lax.axis_index("c")
        base = wid * TPW
        pltpu.sync_copy(dest_hbm.at[0, pl.ds(base, TPW)], idx0_v)
        pltpu.sync_copy(dest_hbm.at[1, pl.ds(base, TPW)], idx1_v)

        bufs = [(a0_v, a1_v, ia0_v, ia1_v, sem_a, sem_wa),
                (b0_v, b1_v, ib0_v, ib1_v, sem_b, sem_wb)]

        def start(c):
            y0, y1, i0, i1, sg, _ = bufs[c % 2]
            for q in range(CC // 16):
                i0[pl.ds(q * 16, 16)] = idx0_v[pl.ds(c * CC + q * 16, 16)]
                i1[pl.ds(q * 16, 16)] = idx1_v[pl.ds(c * CC + q * 16, 16)]
            g0 = pltpu.async_copy(yg_hbm.at[i0], y0, sg)
            g1 = pltpu.async_copy(yg_hbm.at[i1], y1, sg)
            return g0, g1

        pend = start(0)
        wpend = [None, None]
        for c in range(NCH):
            y0, y1, _, _, _, sw = bufs[c % 2]
            pend[0].wait()
            pend[1].wait()
            if c + 1 < NCH:
                for p in wpend[(c + 1) % 2] or ():
                    p.wait()
                wpend[(c + 1) % 2] = None
                pend = start(c + 1)
            rows = pl.ds(base + c * CC, CC)
            s0 = pltpu.async_copy(y0, y0_hbm.at[rows], sw)
            s1 = pltpu.async_copy(y1, y1_hbm.at[rows], sw)
            wpend[c % 2] = (s0, s1)
        for pair in wpend:
            for p in pair or ():
                p.wait()

    return k


def _fin_body(w_ref, y0_ref, y1_ref, out_ref):
    w0 = w_ref[0, :, 0:1]
    w1 = w_ref[1, :, 0:1]
    v0 = y0_ref[...]
    v1 = y1_ref[...]
    lo0 = lax.bitcast_convert_type(lax.shift_left(v0, 16), jnp.float32)
    hi0 = lax.bitcast_convert_type(v0 & MASK_HI, jnp.float32)
    lo1 = lax.bitcast_convert_type(lax.shift_left(v1, 16), jnp.float32)
    hi1 = lax.bitcast_convert_type(v1 & MASK_HI, jnp.float32)
    out_ref[...] = jnp.concatenate(
        [w0 * lo0 + w1 * lo1, w0 * hi0 + w1 * hi1], axis=1)


def _fin_call(wts, y0, y1):
    return pl.pallas_call(
        _fin_body,
        out_shape=jax.ShapeDtypeStruct((T, HIDDEN), jnp.float32),
    )(wts, y0, y1)


def kernel(hidden_states, gate_w, w_gate, w_up, w_down, num_global_tokens,
           max_num_tokens_per_gpu):
    del num_global_tokens, max_num_tokens_per_gpu
    dest, meta, wts, xp = _router_call(hidden_states, gate_w)
    xg = _make_sc_scatter()(xp, dest)
    yg = _ffn_call(meta, xg, w_gate, w_up, w_down)
    y0, y1 = _make_sc_gather()(yg, dest)
    return _fin_call(wts, y0, y1)


# speedup vs baseline: 1.0572x; 1.0572x over previous
import functools

import jax
import jax.numpy as jnp
from jax import lax
from jax.experimental import pallas as pl
from jax.experimental.pallas import tpu as pltpu
from jax.experimental.pallas import tpu_sc as plsc

E = 16
TOP_K = 2
HIDDEN = 1024
INTER = 512
T = 2048
NEG_INF = float("-inf")
HH = HIDDEN // 2
MASK_HI = -65536

BLK = 512
NB = (T * TOP_K) // BLK + (E - 1)
GROUP_ROWS = NB * BLK
CHUNK = 256
NW = 32
TPW = T // NW
CC = 32
NCH = TPW // CC


def _router_body(x_ref, gate_ref, dest_ref, meta_ref, wts_ref, xp_ref):
    x = x_ref[...]
    logits = lax.dot_general(x, gate_ref[...], (((1,), (1,)), ((), ())),
                             preferred_element_type=jnp.float32)
    ii = lax.broadcasted_iota(jnp.int32, (T, E), 1)
    m1 = jnp.max(logits, axis=-1, keepdims=True)
    i1 = jnp.min(jnp.where(logits == m1, ii, E), axis=-1, keepdims=True)
    l2 = jnp.where(ii == i1, NEG_INF, logits)
    m2 = jnp.max(l2, axis=-1, keepdims=True)
    i2 = jnp.min(jnp.where(l2 == m2, ii, E), axis=-1, keepdims=True)
    r = jnp.exp(m2 - m1)
    w1 = 1.0 / (1.0 + r)
    w2 = 1.0 - w1
    wts_ref[0:T, :] = jnp.broadcast_to(w1, (T, 16))
    wts_ref[T:2 * T, :] = jnp.broadcast_to(w2, (T, 16))

    lo = lax.bitcast_convert_type(x[:, :HH], jnp.int32) + 32768
    hi = lax.bitcast_convert_type(x[:, HH:], jnp.int32) + 32768
    xp_ref[...] = lax.shift_right_logical(lo, 16) | (hi & MASK_HI)

    oh0 = (ii == i1).astype(jnp.float32)
    oh1 = (ii == i2).astype(jnp.float32)

    rr = lax.broadcasted_iota(jnp.int32, (CHUNK, CHUNK), 0)
    cc = lax.broadcasted_iota(jnp.int32, (CHUNK, CHUNK), 1)
    ltri = (rr > cc).astype(jnp.float32)

    def _excl_cumsum(oh):
        parts = []
        off = jnp.zeros((1, E), jnp.float32)
        for c in range(T // CHUNK):
            blk = oh[c * CHUNK:(c + 1) * CHUNK, :]
            exc = lax.dot_general(ltri, blk, (((1,), (0,)), ((), ())),
                                  preferred_element_type=jnp.float32)
            parts.append(exc + off)
            off = off + jnp.sum(blk, axis=0, keepdims=True)
        return jnp.concatenate(parts, axis=0), off

    exc0, cnt0 = _excl_cumsum(oh0)
    exc1, cnt1 = _excl_cumsum(oh1)
    counts = cnt0 + cnt1
    nb = jnp.floor((counts + (BLK - 1)) / BLK)
    er = lax.broadcasted_iota(jnp.int32, (E, E), 0)
    ec = lax.broadcasted_iota(jnp.int32, (E, E), 1)
    eutri = (er < ec).astype(jnp.float32)
    bs = lax.dot_general(nb, eutri, (((1,), (0,)), ((), ())),
                         preferred_element_type=jnp.float32)
    total = jnp.sum(nb)

    rank0 = jnp.sum(oh0 * exc0, axis=1)
    rank1 = jnp.sum(oh1 * (cnt0 + exc1), axis=1)
    base0 = jnp.sum(oh0 * bs, axis=1) * BLK
    base1 = jnp.sum(oh1 * bs, axis=1) * BLK
    dest_ref[0, :] = (base0 + rank0).astype(jnp.int32)
    dest_ref[1, :] = (base1 + rank1).astype(jnp.int32)

    bb = lax.broadcasted_iota(jnp.int32, (64, E), 0).astype(jnp.float32)
    emap = jnp.sum((bs <= bb).astype(jnp.int32), axis=1) - 1
    bidx = lax.broadcasted_iota(jnp.int32, (64, 1), 0).astype(jnp.float32)[:, 0]
    active = (bidx < total).astype(jnp.int32)
    meta_ref[0, :] = jnp.clip(emap, 0, E - 1)
    meta_ref[1, :] = active
    meta_ref[2, :] = jnp.minimum(bidx, total - 1.0).astype(jnp.int32)


def _router_call(x, gate_w):
    return pl.pallas_call(
        _router_body,
        out_shape=(
            jax.ShapeDtypeStruct((2, T), jnp.int32),
            jax.ShapeDtypeStruct((3, 64), jnp.int32),
            jax.ShapeDtypeStruct((2 * T, 16), jnp.float32),
            jax.ShapeDtypeStruct((T, HH), jnp.int32),
        ),
    )(x, gate_w)


def _make_sc_scatter():
    mesh = plsc.VectorSubcoreMesh(core_axis_name="c", subcore_axis_name="s")

    @functools.partial(
        pl.kernel, mesh=mesh,
        out_type=jax.ShapeDtypeStruct((GROUP_ROWS, HH), jnp.int32),
        scratch_types=[
            pltpu.VMEM((TPW, HH), jnp.int32),
            pltpu.VMEM((TPW,), jnp.int32),
            pltpu.VMEM((TPW,), jnp.int32),
            pltpu.SemaphoreType.DMA,
            pltpu.SemaphoreType.DMA,
        ],
    )
    def k(xp_hbm, dest_hbm, xg_hbm, rows_v, idx0_v, idx1_v, sem_a, sem_b):
        wid = lax.axis_index("s") * 2 + lax.axis_index("c")
        base = wid * TPW
        pltpu.sync_copy(dest_hbm.at[0, pl.ds(base, TPW)], idx0_v)
        pltpu.sync_copy(dest_hbm.at[1, pl.ds(base, TPW)], idx1_v)
        pltpu.sync_copy(xp_hbm.at[pl.ds(base, TPW)], rows_v)
        c0 = pltpu.async_copy(rows_v, xg_hbm.at[idx0_v], sem_a)
        c1 = pltpu.async_copy(rows_v, xg_hbm.at[idx1_v], sem_b)
        c0.wait()
        c1.wait()

    return k


def _ffn_body(s_ref, xg_ref, wg_ref, wu_ref, wd_ref, yg_ref):
    b = pl.program_id(0)

    @pl.when(s_ref[1, b] == 1)
    def _compute():
        v = xg_ref[...]
        xb = jnp.concatenate(
            [lax.bitcast_convert_type(lax.shift_left(v, 16), jnp.float32),
             lax.bitcast_convert_type(v & MASK_HI, jnp.float32)], axis=1)
        g = lax.dot_general(xb, wg_ref[0], (((1,), (1,)), ((), ())),
                            preferred_element_type=jnp.float32)
        u = lax.dot_general(xb, wu_ref[0], (((1,), (1,)), ((), ())),
                            preferred_element_type=jnp.float32)
        h = (g * jax.nn.sigmoid(g)) * u
        y = lax.dot_general(h, wd_ref[0], (((1,), (1,)), ((), ())),
                            preferred_element_type=jnp.float32)
        ylo = lax.bitcast_convert_type(y[:, :HH], jnp.int32) + 32768
        yhi = lax.bitcast_convert_type(y[:, HH:], jnp.int32) + 32768
        yg_ref[...] = lax.shift_right_logical(ylo, 16) | (yhi & MASK_HI)


def _ffn_call(meta, xg, w_gate, w_up, w_down):
    grid_spec = pltpu.PrefetchScalarGridSpec(
        num_scalar_prefetch=1,
        grid=(NB,),
        in_specs=[
            pl.BlockSpec((BLK, HH), lambda b, s: (s[2, b], 0)),
            pl.BlockSpec((1, INTER, HIDDEN), lambda b, s: (s[0, b], 0, 0)),
            pl.BlockSpec((1, INTER, HIDDEN), lambda b, s: (s[0, b], 0, 0)),
            pl.BlockSpec((1, HIDDEN, INTER), lambda b, s: (s[0, b], 0, 0)),
        ],
        out_specs=pl.BlockSpec((BLK, HH), lambda b, s: (s[2, b], 0)),
    )
    return pl.pallas_call(
        _ffn_body,
        grid_spec=grid_spec,
        out_shape=jax.ShapeDtypeStruct((GROUP_ROWS, HH), jnp.int32),
    )(meta, xg, w_gate, w_up, w_down)


def _make_sc_gather():
    mesh = plsc.VectorSubcoreMesh(core_axis_name="c", subcore_axis_name="s")

    @functools.partial(
        pl.kernel, mesh=mesh,
        out_type=(
            jax.ShapeDtypeStruct((T, HH), jnp.int32),
            jax.ShapeDtypeStruct((T, HH), jnp.int32),
        ),
        scratch_types=[
            pltpu.VMEM((CC, HH), jnp.int32),
            pltpu.VMEM((CC, HH), jnp.int32),
            pltpu.VMEM((CC, HH), jnp.int32),
            pltpu.VMEM((CC, HH), jnp.int32),
            pltpu.VMEM((TPW,), jnp.int32),
            pltpu.VMEM((TPW,), jnp.int32),
            pltpu.VMEM((CC,), jnp.int32),
            pltpu.VMEM((CC,), jnp.int32),
            pltpu.VMEM((CC,), jnp.int32),
            pltpu.VMEM((CC,), jnp.int32),
            pltpu.SemaphoreType.DMA,
            pltpu.SemaphoreType.DMA,
            pltpu.SemaphoreType.DMA,
            pltpu.SemaphoreType.DMA,
        ],
    )
    def k(yg_hbm, dest_hbm, y0_hbm, y1_hbm,
          a0_v, a1_v, b0_v, b1_v, idx0_v, idx1_v,
          ia0_v, ia1_v, ib0_v, ib1_v,
          sem_a, sem_b, sem_wa, sem_wb):
        wid = lax.axis_index("s") * 2 + lax.axis_index("c")
        base = wid * TPW
        pltpu.sync_copy(dest_hbm.at[0, pl.ds(base, TPW)], idx0_v)
        pltpu.sync_copy(dest_hbm.at[1, pl.ds(base, TPW)], idx1_v)

        bufs = [(a0_v, a1_v, ia0_v, ia1_v, sem_a, sem_wa),
                (b0_v, b1_v, ib0_v, ib1_v, sem_b, sem_wb)]

        def start(c):
            y0, y1, i0, i1, sg, _ = bufs[c % 2]
            for q in range(CC // 16):
                i0[pl.ds(q * 16, 16)] = idx0_v[pl.ds(c * CC + q * 16, 16)]
                i1[pl.ds(q * 16, 16)] = idx1_v[pl.ds(c * CC + q * 16, 16)]
            g0 = pltpu.async_copy(yg_hbm.at[i0], y0, sg)
            g1 = pltpu.async_copy(yg_hbm.at[i1], y1, sg)
            return g0, g1

        pend = start(0)
        wpend = [None, None]
        for c in range(NCH):
            y0, y1, _, _, _, sw = bufs[c % 2]
            pend[0].wait()
            pend[1].wait()
            if c + 1 < NCH:
                for p in wpend[(c + 1) % 2] or ():
                    p.wait()
                wpend[(c + 1) % 2] = None
                pend = start(c + 1)
            rows = pl.ds(base + c * CC, CC)
            s0 = pltpu.async_copy(y0, y0_hbm.at[rows], sw)
            s1 = pltpu.async_copy(y1, y1_hbm.at[rows], sw)
            wpend[c % 2] = (s0, s1)
        for pair in wpend:
            for p in pair or ():
                p.wait()

    return k


def _fin_body(w_ref, y0_ref, y1_ref, out_ref):
    w0 = w_ref[0:T, 0:1]
    w1 = w_ref[T:2 * T, 0:1]
    v0 = y0_ref[...]
    v1 = y1_ref[...]
    lo0 = lax.bitcast_convert_type(lax.shift_left(v0, 16), jnp.float32)
    hi0 = lax.bitcast_convert_type(v0 & MASK_HI, jnp.float32)
    lo1 = lax.bitcast_convert_type(lax.shift_left(v1, 16), jnp.float32)
    hi1 = lax.bitcast_convert_type(v1 & MASK_HI, jnp.float32)
    out_ref[...] = jnp.concatenate(
        [w0 * lo0 + w1 * lo1, w0 * hi0 + w1 * hi1], axis=1)


def _fin_call(wts, y0, y1):
    return pl.pallas_call(
        _fin_body,
        out_shape=jax.ShapeDtypeStruct((T, HIDDEN), jnp.float32),
    )(wts, y0, y1)


def kernel(hidden_states, gate_w, w_gate, w_up, w_down, num_global_tokens,
           max_num_tokens_per_gpu):
    del num_global_tokens, max_num_tokens_per_gpu
    dest, meta, wts, xp = _router_call(hidden_states, gate_w)
    xg = _make_sc_scatter()(xp, dest)
    yg = _ffn_call(meta, xg, w_gate, w_up, w_down)
    y0, y1 = _make_sc_gather()(yg, dest)
    return _fin_call(wts, y0, y1)
